# baseline (speedup 1.0000x reference)
import functools

import jax
import jax.numpy as jnp
from jax import lax
from jax.experimental import pallas as pl
from jax.experimental.pallas import tpu as pltpu
from jax.experimental.pallas import tpu_sc as plsc

D = 32
F = 32
B = 16384
U_TAB = 1000000
_FOLD = 4
DF = D * _FOLD
_RT = U_TAB // _FOLD

_UBLK = 16384
_QB = _UBLK // _FOLD
_NBLK = pl.cdiv(U_TAB, _UBLK)
_RT_OUT = _NBLK * _QB


def _fold_body(src_ref, eye_ref, dst_ref):
    x = src_ref[...]
    eye = eye_ref[...]
    parts = [
        jax.lax.dot_general(x[:, j * _QB:(j + 1) * _QB], eye,
                            (((0,), (0,)), ((), ())),
                            preferred_element_type=jnp.float32,
                            precision=jax.lax.Precision.HIGHEST)
        for j in range(_FOLD)
    ]
    dst_ref[...] = jnp.concatenate(parts, axis=1)


def _fold_call(embT, eye):
    return pl.pallas_call(
        _fold_body,
        grid=(_NBLK,),
        in_specs=[pl.BlockSpec((D, _UBLK), lambda i: (0, i)),
                  pl.BlockSpec((D, D), lambda i: (0, 0))],
        out_specs=pl.BlockSpec((_QB, DF), lambda i: (i, 0)),
        out_shape=jax.ShapeDtypeStruct((_RT_OUT, DF), jnp.float32),
        compiler_params=pltpu.CompilerParams(
            dimension_semantics=("arbitrary",)),
    )(embT, eye)

_NC, _NS = 2, 16
_NW = _NC * _NS
_BPW = B // _NW
_HALF = _BPW // 2


def _sc_gather(uidsf, iidsf, uids, iids, uemb4, iemb4, ult, ilt,
               out_u4, out_i4, out_ult, out_ilt,
               uidx_v, iidx_v, uf_v, if_v, urows_v, irows_v, ultv, iltv,
               sem, semt):
    wid = lax.axis_index("s") * _NC + lax.axis_index("c")
    base = wid * _BPW
    pltpu.sync_copy(uids.at[pl.ds(base, _BPW)], uidx_v)
    pltpu.sync_copy(iids.at[pl.ds(base, _BPW)], iidx_v)
    ct1 = pltpu.async_copy(ult.at[uidx_v], ultv, semt)
    ct2 = pltpu.async_copy(ilt.at[iidx_v], iltv, semt)
    for p in range(2):
        off = base + p * _HALF
        pltpu.sync_copy(uidsf.at[pl.ds(off, _HALF)], uf_v)
        pltpu.sync_copy(iidsf.at[pl.ds(off, _HALF)], if_v)
        c1 = pltpu.async_copy(uemb4.at[uf_v], urows_v, sem)
        c2 = pltpu.async_copy(iemb4.at[if_v], irows_v, sem)
        c1.wait()
        c2.wait()
        pltpu.sync_copy(urows_v, out_u4.at[pl.ds(off, _HALF)])
        pltpu.sync_copy(irows_v, out_i4.at[pl.ds(off, _HALF)])
    ct1.wait()
    ct2.wait()
    pltpu.sync_copy(ultv, out_ult.at[pl.ds(base, _BPW)])
    pltpu.sync_copy(iltv, out_ilt.at[pl.ds(base, _BPW)])


_gather_call = functools.partial(
    pl.kernel,
    mesh=plsc.VectorSubcoreMesh(core_axis_name="c", subcore_axis_name="s",
                                num_cores=_NC, num_subcores=_NS),
    out_type=[
        jax.ShapeDtypeStruct((B, DF), jnp.float32),
        jax.ShapeDtypeStruct((B, DF), jnp.float32),
        jax.ShapeDtypeStruct((B,), jnp.float32),
        jax.ShapeDtypeStruct((B,), jnp.float32),
    ],
    scratch_types=[
        pltpu.VMEM((_BPW,), jnp.int32),
        pltpu.VMEM((_BPW,), jnp.int32),
        pltpu.VMEM((_HALF,), jnp.int32),
        pltpu.VMEM((_HALF,), jnp.int32),
        pltpu.VMEM((_HALF, DF), jnp.float32),
        pltpu.VMEM((_HALF, DF), jnp.float32),
        pltpu.VMEM((_BPW,), jnp.float32),
        pltpu.VMEM((_BPW,), jnp.float32),
        pltpu.SemaphoreType.DMA,
        pltpu.SemaphoreType.DMA,
    ],
    compiler_params=pltpu.CompilerParams(use_tc_tiling_on_sc=False),
)(_sc_gather)

_BB = 2048


def _dense_body(qt_ref, ts_ref, ult_ref, ilt_ref, uid_ref, iid_ref,
                u4_ref, i4_ref, f_ref, tpw_ref, w_ref, b_ref,
                nu_ref, ni_ref, pred_ref):
    tpw = tpw_ref[...]
    ts = ts_ref[...]
    qt = qt_ref[0, 0]
    u4 = u4_ref[...]
    i4 = i4_ref[...]
    urem = (uid_ref[...] >> 12) & (_FOLD - 1)
    irem = (iid_ref[...] >> 12) & (_FOLD - 1)
    u = jnp.zeros_like(u4[:, :D])
    it = jnp.zeros_like(u)
    for r in range(_FOLD):
        u = u + jnp.where(urem == r, u4[:, r * D:(r + 1) * D], 0.0)
        it = it + jnp.where(irem == r, i4[:, r * D:(r + 1) * D], 0.0)
    us = u * (1.0 + (ts - ult_ref[...]) * tpw)
    isc = it * (1.0 + (ts - ilt_ref[...]) * tpw)
    ps = u * (1.0 + (qt - ult_ref[...]) * tpw)
    z = jnp.concatenate([us, isc, f_ref[...], ps], axis=1)
    out = jnp.dot(z, w_ref[...], preferred_element_type=jnp.float32,
                  precision=jax.lax.Precision.HIGHEST)
    out = out + b_ref[...]
    nu_ref[...] = jnp.tanh(out[:, :D])
    ni_ref[...] = jnp.tanh(out[:, D:2 * D])
    pred_ref[...] = out[:, 2 * D:3 * D]


def _dense_call(qt, ts, ultc, iltc, uid2, iid2, u4, i4, features,
                tpw_row, w_all, b_all):
    grid = (B // _BB,)
    col = lambda ib: (ib, 0)
    fixed = lambda ib: (0, 0)
    return pl.pallas_call(
        _dense_body,
        grid=grid,
        in_specs=[
            pl.BlockSpec(memory_space=pltpu.SMEM),
            pl.BlockSpec((_BB, 1), col),
            pl.BlockSpec((_BB, 1), col),
            pl.BlockSpec((_BB, 1), col),
            pl.BlockSpec((_BB, 1), col),
            pl.BlockSpec((_BB, 1), col),
            pl.BlockSpec((_BB, DF), col),
            pl.BlockSpec((_BB, DF), col),
            pl.BlockSpec((_BB, F), col),
            pl.BlockSpec((1, D), fixed),
            pl.BlockSpec((4 * D, 3 * D), fixed),
            pl.BlockSpec((1, 3 * D), fixed),
        ],
        out_specs=[
            pl.BlockSpec((_BB, D), col),
            pl.BlockSpec((_BB, D), col),
            pl.BlockSpec((_BB, D), col),
        ],
        out_shape=[
            jax.ShapeDtypeStruct((B, D), jnp.float32),
            jax.ShapeDtypeStruct((B, D), jnp.float32),
            jax.ShapeDtypeStruct((B, D), jnp.float32),
        ],
        compiler_params=pltpu.CompilerParams(
            dimension_semantics=("arbitrary",)),
    )(qt, ts, ultc, iltc, uid2, iid2, u4, i4, features, tpw_row, w_all, b_all)


def kernel(user_ids, item_ids, timestamps, features, query_time,
           user_embeddings, item_embeddings, user_last_time, item_last_time,
           time_proj_w, Wp, bp,
           W_ih_u, W_hh_u, b_ih_u, b_hh_u,
           W_ih_i, W_hh_i, b_ih_i, b_hh_i):
    eye = jnp.eye(D, dtype=jnp.float32)
    uemb4 = _fold_call(user_embeddings.T, eye)
    iemb4 = _fold_call(item_embeddings.T, eye)
    uidsf = ((user_ids >> 14) << 12) | (user_ids & 4095)
    iidsf = ((item_ids >> 14) << 12) | (item_ids & 4095)

    u4, i4, ult, ilt = _gather_call(
        uidsf, iidsf, user_ids, item_ids, uemb4, iemb4,
        user_last_time, item_last_time)

    zeros = jnp.zeros((D, D), jnp.float32)
    wu = jnp.concatenate([W_ih_u[:, :D].T + W_hh_u.T,
                          W_ih_u[:, D:2 * D].T,
                          W_ih_u[:, 2 * D:].T,
                          zeros], axis=0)
    wi = jnp.concatenate([W_ih_i[:, D:2 * D].T,
                          W_ih_i[:, :D].T + W_hh_i.T,
                          W_ih_i[:, 2 * D:].T,
                          zeros], axis=0)
    wp = jnp.concatenate([zeros, zeros, zeros, Wp.T], axis=0)
    w_all = jnp.concatenate([wu, wi, wp], axis=1)
    b_all = jnp.concatenate([b_ih_u + b_hh_u, b_ih_i + b_hh_i, bp])[None, :]

    qt = jnp.full((1, 1), query_time, jnp.float32)
    ts = timestamps[:, None]
    tpw_row = time_proj_w.T

    nu, ni, pred = _dense_call(qt, ts, ult[:, None], ilt[:, None],
                               user_ids[:, None], item_ids[:, None],
                               u4, i4, features, tpw_row, w_all, b_all)
    return (pred, nu, ni)


# speedup vs baseline: 1.9060x; 1.9060x over previous
import functools

import jax
import jax.numpy as jnp
from jax import lax
from jax.experimental import pallas as pl
from jax.experimental.pallas import tpu as pltpu
from jax.experimental.pallas import tpu_sc as plsc

D = 32
F = 32
B = 16384
U_TAB = 1000000
_FOLD = 4
DF = D * _FOLD
_RT = U_TAB // _FOLD

_UBLK = 16384
_QB = _UBLK // _FOLD
_NBLK = pl.cdiv(U_TAB, _UBLK)
_RT_OUT = _NBLK * _QB


def _fold_body(src_ref, dst_ref):
    x = src_ref[...]
    parts = [jnp.transpose(x[:, j * _QB:(j + 1) * _QB]) for j in range(_FOLD)]
    dst_ref[...] = jnp.concatenate(parts, axis=1)


def _fold_call(embT):
    return pl.pallas_call(
        _fold_body,
        grid=(_NBLK,),
        in_specs=[pl.BlockSpec((D, _UBLK), lambda i: (0, i))],
        out_specs=pl.BlockSpec((_QB, DF), lambda i: (i, 0)),
        out_shape=jax.ShapeDtypeStruct((_RT_OUT, DF), jnp.float32),
        compiler_params=pltpu.CompilerParams(
            dimension_semantics=("arbitrary",)),
    )(embT)

_NC, _NS = 2, 16
_NW = _NC * _NS
_BPW = B // _NW
_HALF = _BPW // 2


def _sc_gather(uidsf, iidsf, uids, iids, uemb4, iemb4, ult, ilt,
               out_u4, out_i4, out_ult, out_ilt,
               uidx_v, iidx_v, uf_v, if_v, urows_v, irows_v, ultv, iltv,
               sem, semt):
    wid = lax.axis_index("s") * _NC + lax.axis_index("c")
    base = wid * _BPW
    pltpu.sync_copy(uids.at[pl.ds(base, _BPW)], uidx_v)
    pltpu.sync_copy(iids.at[pl.ds(base, _BPW)], iidx_v)
    ct1 = pltpu.async_copy(ult.at[uidx_v], ultv, semt)
    ct2 = pltpu.async_copy(ilt.at[iidx_v], iltv, semt)
    for p in range(2):
        off = base + p * _HALF
        pltpu.sync_copy(uidsf.at[pl.ds(off, _HALF)], uf_v)
        pltpu.sync_copy(iidsf.at[pl.ds(off, _HALF)], if_v)
        c1 = pltpu.async_copy(uemb4.at[uf_v], urows_v, sem)
        c2 = pltpu.async_copy(iemb4.at[if_v], irows_v, sem)
        c1.wait()
        c2.wait()
        pltpu.sync_copy(urows_v, out_u4.at[pl.ds(off, _HALF)])
        pltpu.sync_copy(irows_v, out_i4.at[pl.ds(off, _HALF)])
    ct1.wait()
    ct2.wait()
    pltpu.sync_copy(ultv, out_ult.at[pl.ds(base, _BPW)])
    pltpu.sync_copy(iltv, out_ilt.at[pl.ds(base, _BPW)])


_gather_call = functools.partial(
    pl.kernel,
    mesh=plsc.VectorSubcoreMesh(core_axis_name="c", subcore_axis_name="s",
                                num_cores=_NC, num_subcores=_NS),
    out_type=[
        jax.ShapeDtypeStruct((B, DF), jnp.float32),
        jax.ShapeDtypeStruct((B, DF), jnp.float32),
        jax.ShapeDtypeStruct((B,), jnp.float32),
        jax.ShapeDtypeStruct((B,), jnp.float32),
    ],
    scratch_types=[
        pltpu.VMEM((_BPW,), jnp.int32),
        pltpu.VMEM((_BPW,), jnp.int32),
        pltpu.VMEM((_HALF,), jnp.int32),
        pltpu.VMEM((_HALF,), jnp.int32),
        pltpu.VMEM((_HALF, DF), jnp.float32),
        pltpu.VMEM((_HALF, DF), jnp.float32),
        pltpu.VMEM((_BPW,), jnp.float32),
        pltpu.VMEM((_BPW,), jnp.float32),
        pltpu.SemaphoreType.DMA,
        pltpu.SemaphoreType.DMA,
    ],
    compiler_params=pltpu.CompilerParams(use_tc_tiling_on_sc=False),
)(_sc_gather)

_BB = 2048


def _dense_body(qt_ref, ts_ref, ult_ref, ilt_ref, uid_ref, iid_ref,
                u4_ref, i4_ref, f_ref, tpw_ref, w_ref, b_ref,
                nu_ref, ni_ref, pred_ref):
    tpw = tpw_ref[...]
    ts = jnp.transpose(ts_ref[...])
    ult = jnp.transpose(ult_ref[...])
    ilt = jnp.transpose(ilt_ref[...])
    qt = qt_ref[0, 0]
    u4 = u4_ref[...]
    i4 = i4_ref[...]
    urem = (jnp.transpose(uid_ref[...]) >> 12) & (_FOLD - 1)
    irem = (jnp.transpose(iid_ref[...]) >> 12) & (_FOLD - 1)
    u = jnp.zeros_like(u4[:, :D])
    it = jnp.zeros_like(u)
    for r in range(_FOLD):
        u = u + jnp.where(urem == r, u4[:, r * D:(r + 1) * D], 0.0)
        it = it + jnp.where(irem == r, i4[:, r * D:(r + 1) * D], 0.0)
    us = u * (1.0 + (ts - ult) * tpw)
    isc = it * (1.0 + (ts - ilt) * tpw)
    ps = u * (1.0 + (qt - ult) * tpw)
    z = jnp.concatenate([us, isc, f_ref[...], ps], axis=1)
    out = jnp.dot(z, w_ref[...], preferred_element_type=jnp.float32,
                  precision=jax.lax.Precision.HIGHEST)
    out = out + b_ref[...]
    nu_ref[...] = jnp.tanh(out[:, :D])
    ni_ref[...] = jnp.tanh(out[:, D:2 * D])
    pred_ref[...] = out[:, 2 * D:3 * D]


def _dense_call(qt, ts, ultc, iltc, uid2, iid2, u4, i4, features,
                tpw_row, w_all, b_all):
    grid = (B // _BB,)
    col = lambda ib: (ib, 0)
    fixed = lambda ib: (0, 0)
    return pl.pallas_call(
        _dense_body,
        grid=grid,
        in_specs=[
            pl.BlockSpec(memory_space=pltpu.SMEM),
            pl.BlockSpec((1, _BB), lambda ib: (0, ib)),
            pl.BlockSpec((1, _BB), lambda ib: (0, ib)),
            pl.BlockSpec((1, _BB), lambda ib: (0, ib)),
            pl.BlockSpec((1, _BB), lambda ib: (0, ib)),
            pl.BlockSpec((1, _BB), lambda ib: (0, ib)),
            pl.BlockSpec((_BB, DF), col),
            pl.BlockSpec((_BB, DF), col),
            pl.BlockSpec((_BB, F), col),
            pl.BlockSpec((1, D), fixed),
            pl.BlockSpec((4 * D, 3 * D), fixed),
            pl.BlockSpec((1, 3 * D), fixed),
        ],
        out_specs=[
            pl.BlockSpec((_BB, D), col),
            pl.BlockSpec((_BB, D), col),
            pl.BlockSpec((_BB, D), col),
        ],
        out_shape=[
            jax.ShapeDtypeStruct((B, D), jnp.float32),
            jax.ShapeDtypeStruct((B, D), jnp.float32),
            jax.ShapeDtypeStruct((B, D), jnp.float32),
        ],
        compiler_params=pltpu.CompilerParams(
            dimension_semantics=("arbitrary",)),
    )(qt, ts, ultc, iltc, uid2, iid2, u4, i4, features, tpw_row, w_all, b_all)


def kernel(user_ids, item_ids, timestamps, features, query_time,
           user_embeddings, item_embeddings, user_last_time, item_last_time,
           time_proj_w, Wp, bp,
           W_ih_u, W_hh_u, b_ih_u, b_hh_u,
           W_ih_i, W_hh_i, b_ih_i, b_hh_i):
    uemb4 = _fold_call(user_embeddings.T)
    iemb4 = _fold_call(item_embeddings.T)
    uidsf = ((user_ids >> 14) << 12) | (user_ids & 4095)
    iidsf = ((item_ids >> 14) << 12) | (item_ids & 4095)

    u4, i4, ult, ilt = _gather_call(
        uidsf, iidsf, user_ids, item_ids, uemb4, iemb4,
        user_last_time, item_last_time)

    zeros = jnp.zeros((D, D), jnp.float32)
    wu = jnp.concatenate([W_ih_u[:, :D].T + W_hh_u.T,
                          W_ih_u[:, D:2 * D].T,
                          W_ih_u[:, 2 * D:].T,
                          zeros], axis=0)
    wi = jnp.concatenate([W_ih_i[:, D:2 * D].T,
                          W_ih_i[:, :D].T + W_hh_i.T,
                          W_ih_i[:, 2 * D:].T,
                          zeros], axis=0)
    wp = jnp.concatenate([zeros, zeros, zeros, Wp.T], axis=0)
    w_all = jnp.concatenate([wu, wi, wp], axis=1)
    b_all = jnp.concatenate([b_ih_u + b_hh_u, b_ih_i + b_hh_i, bp])[None, :]

    qt = jnp.full((1, 1), query_time, jnp.float32)
    ts = timestamps[None, :]
    tpw_row = time_proj_w.T

    nu, ni, pred = _dense_call(qt, ts, ult[None, :], ilt[None, :],
                               user_ids[None, :], item_ids[None, :],
                               u4, i4, features, tpw_row, w_all, b_all)
    return (pred, nu, ni)
